# baseline (speedup 1.0000x reference)
import functools

import jax
import jax.numpy as jnp
from jax import lax
from jax.experimental import pallas as pl
from jax.experimental.pallas import tpu as pltpu
from jax.experimental.pallas import tpu_sc as plsc

_CP = 128


def _gather_half(idxs_flat, table, N):
    RT = table.shape[0]

    GR = 400
    assert RT % GR == 0
    NGROUPS = RT // GR

    info = plsc.get_sparse_core_info()
    NC, NS = info.num_cores, info.num_subcores
    NW = NC * NS
    ITERS = -(-NGROUPS // NW)

    mesh = plsc.VectorSubcoreMesh(core_axis_name="c", subcore_axis_name="s")

    @functools.partial(
        pl.kernel,
        mesh=mesh,
        out_type=jax.ShapeDtypeStruct((RT, _CP), jnp.float32),
        scratch_types=[
            pltpu.VMEM((GR,), jnp.int32),
            pltpu.VMEM((GR,), jnp.int32),
            pltpu.VMEM((GR, _CP), jnp.float32),
            pltpu.VMEM((GR, _CP), jnp.float32),
            pltpu.SemaphoreType.DMA,
            pltpu.SemaphoreType.DMA,
            pltpu.SemaphoreType.DMA,
            pltpu.SemaphoreType.DMA,
            pltpu.SemaphoreType.DMA,
            pltpu.SemaphoreType.DMA,
        ],
        compiler_params=pltpu.CompilerParams(use_tc_tiling_on_sc=False),
    )
    def k(idx_hbm, tab_hbm, out_hbm,
          ib0, ib1, db0, db1, si0, si1, sg0, sg1, sw0, sw1):
        wid = lax.axis_index("s") * NC + lax.axis_index("c")
        IB, DB = (ib0, ib1), (db0, db1)
        SI, SG, SW = (si0, si1), (sg0, sg1), (sw0, sw1)

        def gid(t):
            return wid + NW * t

        def stage(t, u):
            s, o = u, 1 - u

            @pl.when(jnp.logical_and(t >= 2, gid(t - 2) < NGROUPS))
            def _():
                pltpu.make_async_copy(
                    DB[s], out_hbm.at[pl.ds(gid(t - 2) * GR, GR)], SW[s]).wait()

            @pl.when(jnp.logical_and(
                jnp.logical_and(t >= 1, t <= ITERS), gid(t - 1) < NGROUPS))
            def _():
                pltpu.make_async_copy(tab_hbm.at[IB[o]], DB[o], SG[o]).wait()
                pltpu.async_copy(
                    DB[o], out_hbm.at[pl.ds(gid(t - 1) * GR, GR)], SW[o])

            @pl.when(jnp.logical_and(t + 1 <= ITERS - 1, gid(t + 1) < NGROUPS))
            def _():
                pltpu.async_copy(
                    idx_hbm.at[pl.ds(gid(t + 1) * GR, GR)], IB[o], SI[o])

            @pl.when(jnp.logical_and(t <= ITERS - 1, gid(t) < NGROUPS))
            def _():
                g_row = gid(t) * GR
                pltpu.make_async_copy(
                    idx_hbm.at[pl.ds(g_row, GR)], IB[s], SI[s]).wait()

                def conv(i, carry):
                    off = pl.multiple_of(i * 16, 16)
                    pos = g_row + off + lax.iota(jnp.int32, 16)
                    v = IB[s][pl.ds(off, 16)] + lax.div(pos, N) * N
                    pltpu.async_copy(
                        tab_hbm.at[v], DB[s].at[pl.ds(off, 16)], SG[s])
                    return carry

                lax.fori_loop(0, GR // 16, conv, 0)

        pltpu.async_copy(idx_hbm.at[pl.ds(gid(0) * GR, GR)], IB[0], SI[0])

        def body(tt, carry):
            stage(2 * tt, 0)
            stage(2 * tt + 1, 1)
            return carry

        lax.fori_loop(0, (ITERS + 3) // 2, body, 0)

    return k(idxs_flat, table)


def kernel(idxs, locs, data):
    B, N, D = locs.shape
    C = data.shape[2]
    BH = B // 2

    outs = []
    for h in range(2):
        RT2 = BH * N
        bsl = slice(h * BH, (h + 1) * BH)
        table = jnp.pad(
            jnp.concatenate(
                [data[bsl].reshape(RT2, C), locs[bsl].reshape(RT2, D)],
                axis=1),
            ((0, 0), (0, _CP - C - D)))
        outs.append(_gather_half(idxs[bsl].reshape(RT2), table, N))

    out = jnp.concatenate(outs, axis=0)
    RT = B * N
    nlocs = out[:, C:C + D].reshape(B, N, D)
    ndata = out[:, :C].reshape(B, N, C)
    return (nlocs, ndata)


# speedup vs baseline: 1.3229x; 1.3229x over previous
import functools

import jax
import jax.numpy as jnp
from jax import lax
from jax.experimental import pallas as pl
from jax.experimental.pallas import tpu as pltpu
from jax.experimental.pallas import tpu_sc as plsc

_CP = 128


def kernel(idxs, locs, data):
    B, N, D = locs.shape
    C = data.shape[2]
    RT = B * N

    GR = 400
    assert RT % GR == 0
    NGROUPS = RT // GR

    info = plsc.get_sparse_core_info()
    NC, NS = info.num_cores, info.num_subcores
    NW = NC * NS
    ITERS = -(-NGROUPS // NW)

    idxs_flat = idxs.reshape(RT)
    table = jnp.pad(
        jnp.concatenate([data.reshape(RT, C), locs.reshape(RT, D)], axis=1),
        ((0, 0), (0, _CP - C - D)))

    mesh = plsc.VectorSubcoreMesh(core_axis_name="c", subcore_axis_name="s")

    @functools.partial(
        pl.kernel,
        mesh=mesh,
        out_type=jax.ShapeDtypeStruct((RT, _CP), jnp.float32),
        scratch_types=[
            pltpu.VMEM((GR,), jnp.int32),
            pltpu.VMEM((GR,), jnp.int32),
            pltpu.VMEM((GR, _CP), jnp.float32),
            pltpu.VMEM((GR, _CP), jnp.float32),
            pltpu.SemaphoreType.DMA,
            pltpu.SemaphoreType.DMA,
            pltpu.SemaphoreType.DMA,
            pltpu.SemaphoreType.DMA,
            pltpu.SemaphoreType.DMA,
            pltpu.SemaphoreType.DMA,
        ],
        compiler_params=pltpu.CompilerParams(use_tc_tiling_on_sc=False),
    )
    def k(idx_hbm, tab_hbm, out_hbm,
          ib0, ib1, db0, db1, si0, si1, sg0, sg1, sw0, sw1):
        wid = lax.axis_index("s") * NC + lax.axis_index("c")
        IB, DB = (ib0, ib1), (db0, db1)
        SI, SG, SW = (si0, si1), (sg0, sg1), (sw0, sw1)

        def gid(t):
            return wid + NW * t

        def stage(t, u):
            s, o = u, 1 - u

            @pl.when(jnp.logical_and(t >= 2, gid(t - 2) < NGROUPS))
            def _():
                pltpu.make_async_copy(
                    DB[s], out_hbm.at[pl.ds(gid(t - 2) * GR, GR)], SW[s]).wait()

            @pl.when(jnp.logical_and(
                jnp.logical_and(t >= 1, t <= ITERS), gid(t - 1) < NGROUPS))
            def _():
                pltpu.make_async_copy(tab_hbm.at[IB[o]], DB[o], SG[o]).wait()
                pltpu.async_copy(
                    DB[o], out_hbm.at[pl.ds(gid(t - 1) * GR, GR)], SW[o])

            @pl.when(jnp.logical_and(t + 1 <= ITERS - 1, gid(t + 1) < NGROUPS))
            def _():
                pltpu.async_copy(
                    idx_hbm.at[pl.ds(gid(t + 1) * GR, GR)], IB[o], SI[o])

            @pl.when(jnp.logical_and(t <= ITERS - 1, gid(t) < NGROUPS))
            def _():
                g_row = gid(t) * GR
                pltpu.make_async_copy(
                    idx_hbm.at[pl.ds(g_row, GR)], IB[s], SI[s]).wait()

                def conv(i, carry):
                    off = pl.multiple_of(i * 16, 16)
                    pos = g_row + off + lax.iota(jnp.int32, 16)
                    v = IB[s][pl.ds(off, 16)] + lax.div(pos, N) * N
                    pltpu.async_copy(
                        tab_hbm.at[v], DB[s].at[pl.ds(off, 16)], SG[s])
                    return carry

                lax.fori_loop(0, GR // 16, conv, 0)

        pltpu.async_copy(idx_hbm.at[pl.ds(gid(0) * GR, GR)], IB[0], SI[0])

        def body(tt, carry):
            stage(2 * tt, 0)
            stage(2 * tt + 1, 1)
            return carry

        lax.fori_loop(0, (ITERS + 3) // 2, body, 0)

    out = k(idxs_flat, table)
    nlocs = out[:, C:C + D].reshape(B, N, D)
    ndata = out[:, :C].reshape(B, N, C)
    return (nlocs, ndata)
